# baseline (speedup 1.0000x reference)
import functools

import jax
import jax.numpy as jnp
from jax.experimental import pallas as pl

RMIN = 0.5
RMAX = 6.0
M2 = 4
ENERGY_SHIFT = -3.37

TILE = 2048


def _dp_tile_kernel(drt_ref, davg_ref, drstd_ref,
                    ew0_ref, eb0_ref, ew1_ref, eb1_ref, ew2_ref, eb2_ref,
                    fw0_ref, fb0_ref, fw1_ref, fb1_ref, fw2_ref, fb2_ref,
                    out_ref, *, K):
    f32 = jnp.float32
    r = drt_ref[0]
    valid = r > 1e-6
    r_safe = jnp.where(valid, r, 1.0)
    uu = (r_safe - RMIN) * (1.0 / (RMAX - RMIN))
    poly = uu ** 3 * (-6.0 * uu ** 2 + 15.0 * uu - 10.0) + 1.0
    s = jnp.where(r_safe < RMIN, 1.0 / r_safe,
                  jnp.where(r_safe < RMAX, poly / r_safe, 0.0))
    s = jnp.where(valid, s, 0.0)
    sr = s / r_safe

    Ri0 = (s - davg_ref[0]) * drstd_ref[0]
    Ri1 = (sr * drt_ref[1] - davg_ref[1]) * drstd_ref[1]
    Ri2 = (sr * drt_ref[2] - davg_ref[2]) * drstd_ref[2]
    Ri3 = (sr * drt_ref[3] - davg_ref[3]) * drstd_ref[3]

    ew1T = ew1_ref[...]
    ew2T = ew2_ref[...]
    eb1c = eb1_ref[...]
    eb2c = eb2_ref[...]
    ew0c = ew0_ref[...]
    eb0c = eb0_ref[...]

    T = r.shape[1]
    GR0 = jnp.zeros((16, T), f32)
    GR1 = jnp.zeros((16, T), f32)
    GR2 = jnp.zeros((16, T), f32)
    GR3 = jnp.zeros((16, T), f32)
    ones = jnp.ones((1, T), f32)
    GRP = 8
    for g in range(K // GRP):
        ks = range(g * GRP, (g + 1) * GRP)
        h0s = [jnp.tanh(ew0c * Ri0[k:k + 1, :] + eb0c) for k in ks]
        resid = jnp.concatenate([h for h0 in h0s for h in (h0, h0)], axis=0)
        H0a = jnp.concatenate(h0s + [ones], axis=0)
        H1 = jnp.tanh(jnp.dot(ew1T, H0a, preferred_element_type=f32)) + resid
        H1a = jnp.concatenate([H1, ones], axis=0)
        Gg = jnp.tanh(jnp.dot(ew2T, H1a, preferred_element_type=f32)) + H1
        for j, k in enumerate(ks):
            G = Gg[16 * j:16 * (j + 1), :]
            GR0 = GR0 + Ri0[k:k + 1, :] * G
            GR1 = GR1 + Ri1[k:k + 1, :] * G
            GR2 = GR2 + Ri2[k:k + 1, :] * G
            GR3 = GR3 + Ri3[k:k + 1, :] * G

    invK = 1.0 / K
    GR0 = GR0 * invK
    GR1 = GR1 * invK
    GR2 = GR2 * invK
    GR3 = GR3 * invK

    dq = []
    for q in range(M2):
        dq.append(GR0 * GR0[q:q + 1, :] + GR1 * GR1[q:q + 1, :]
                  + GR2 * GR2[q:q + 1, :] + GR3 * GR3[q:q + 1, :])
    D64 = jnp.concatenate(dq + [ones], axis=0)

    f0 = jnp.tanh(jnp.dot(fw0_ref[...], D64, preferred_element_type=f32))
    f0a = jnp.concatenate([f0, ones], axis=0)
    f1 = jnp.tanh(jnp.dot(fw1_ref[...], f0a, preferred_element_type=f32)) + f0
    f1a = jnp.concatenate([f1, ones], axis=0)
    Ei = jnp.dot(fw2_ref[...], f1a, preferred_element_type=f32) \
        + ENERGY_SHIFT
    out_ref[...] = Ei


def kernel(list_neigh, Imagetype_map, atom_type, ImageDR, nghost, davg, dstd,
           ew0, eb0, ew1, eb1, ew2, eb2, fw0, fb0, fw1, fb1, fw2, fb2):
    B, N, K, _ = ImageDR.shape
    rm = jnp.where(list_neigh[0] > 0, ImageDR[0, :, :, 0], 0.0)
    dr_m = jnp.concatenate([rm[..., None], ImageDR[0, :, :, 1:]], axis=-1)
    drt = jnp.transpose(dr_m, (2, 1, 0))

    davgT = davg.reshape(K, 4).T.reshape(4, K, 1)
    drstdT = (1.0 / dstd).reshape(K, 4).T.reshape(4, K, 1)

    GRP = 8
    eye = jnp.eye(GRP, dtype=jnp.float32)
    ew0c = ew0.reshape(8, 1)
    eb0c = eb0.reshape(8, 1)
    ew1T = jnp.concatenate(
        [jnp.kron(eye, ew1.T),
         jnp.tile(eb1, GRP).reshape(16 * GRP, 1)], axis=1)
    eb1c = eb1.reshape(16, 1)
    ew2T = jnp.concatenate(
        [jnp.kron(eye, ew2.T),
         jnp.tile(eb2, GRP).reshape(16 * GRP, 1)], axis=1)
    eb2c = eb2.reshape(16, 1)
    fw0T = jnp.concatenate(
        [fw0.reshape(16, 4, 32).transpose(2, 1, 0).reshape(32, 64),
         fb0.reshape(32, 1)], axis=1)
    fb0c = fb0.reshape(32, 1)
    fw1T = jnp.concatenate([fw1.T, fb1.reshape(32, 1)], axis=1)
    fb1c = fb1.reshape(32, 1)
    fw2T = jnp.concatenate([fw2.T, fb2.reshape(1, 1)], axis=1)
    fb2c = fb2.reshape(1, 1)

    grid = (pl.cdiv(N, TILE),)
    rep2 = lambda a: pl.BlockSpec(a.shape, lambda i: (0, 0))
    rep3 = lambda a: pl.BlockSpec(a.shape, lambda i: (0, 0, 0))

    ei = pl.pallas_call(
        functools.partial(_dp_tile_kernel, K=K),
        grid=grid,
        in_specs=[
            pl.BlockSpec((4, K, TILE), lambda i: (0, 0, i)),
            rep3(davgT), rep3(drstdT),
            rep2(ew0c), rep2(eb0c), rep2(ew1T), rep2(eb1c),
            rep2(ew2T), rep2(eb2c), rep2(fw0T), rep2(fb0c),
            rep2(fw1T), rep2(fb1c), rep2(fw2T), rep2(fb2c),
        ],
        out_specs=pl.BlockSpec((1, TILE), lambda i: (0, i)),
        out_shape=jax.ShapeDtypeStruct((1, N), jnp.float32),
    )(drt, davgT, drstdT, ew0c, eb0c, ew1T, eb1c, ew2T, eb2c,
      fw0T, fb0c, fw1T, fb1c, fw2T, fb2c)

    Ei = ei.reshape(B, N)
    Etot = jnp.sum(Ei, axis=1, keepdims=True)
    return Etot, Ei


# speedup vs baseline: 1.2276x; 1.2276x over previous
import functools

import jax
import jax.numpy as jnp
from jax.experimental import pallas as pl

RMIN = 0.5
RMAX = 6.0
M2 = 4
ENERGY_SHIFT = -3.37

TILE = 2048


def _dp_tile_kernel(drt_ref, ngt_ref, davg_ref, drstd_ref,
                    ew0_ref, eb0_ref, ew1_ref, eb1_ref, ew2_ref, eb2_ref,
                    fw0_ref, fb0_ref, fw1_ref, fb1_ref, fw2_ref, fb2_ref,
                    out_ref, *, K):
    f32 = jnp.float32
    r = drt_ref[0]
    valid = (jnp.transpose(ngt_ref[...], (1, 0)) > 0) & (r > 1e-6)
    r_safe = jnp.where(valid, r, 1.0)
    uu = (r_safe - RMIN) * (1.0 / (RMAX - RMIN))
    poly = uu ** 3 * (-6.0 * uu ** 2 + 15.0 * uu - 10.0) + 1.0
    s = jnp.where(r_safe < RMIN, 1.0 / r_safe,
                  jnp.where(r_safe < RMAX, poly / r_safe, 0.0))
    s = jnp.where(valid, s, 0.0)
    sr = s / r_safe

    Ri0 = (s - davg_ref[0]) * drstd_ref[0]
    Ri1 = (sr * drt_ref[1] - davg_ref[1]) * drstd_ref[1]
    Ri2 = (sr * drt_ref[2] - davg_ref[2]) * drstd_ref[2]
    Ri3 = (sr * drt_ref[3] - davg_ref[3]) * drstd_ref[3]

    ew1T = ew1_ref[...]
    ew2T = ew2_ref[...]
    eb1c = eb1_ref[...]
    eb2c = eb2_ref[...]
    ew0c = ew0_ref[...]
    eb0c = eb0_ref[...]

    T = r.shape[1]
    GR0 = jnp.zeros((16, T), f32)
    GR1 = jnp.zeros((16, T), f32)
    GR2 = jnp.zeros((16, T), f32)
    GR3 = jnp.zeros((16, T), f32)
    ones = jnp.ones((1, T), f32)
    GRP = 8
    for g in range(K // GRP):
        ks = range(g * GRP, (g + 1) * GRP)
        h0s = [jnp.tanh(ew0c * Ri0[k:k + 1, :] + eb0c) for k in ks]
        resid = jnp.concatenate([h for h0 in h0s for h in (h0, h0)], axis=0)
        H0a = jnp.concatenate(h0s + [ones], axis=0)
        H1 = jnp.tanh(jnp.dot(ew1T, H0a, preferred_element_type=f32)) + resid
        H1a = jnp.concatenate([H1, ones], axis=0)
        Gg = jnp.tanh(jnp.dot(ew2T, H1a, preferred_element_type=f32)) + H1
        for j, k in enumerate(ks):
            G = Gg[16 * j:16 * (j + 1), :]
            GR0 = GR0 + Ri0[k:k + 1, :] * G
            GR1 = GR1 + Ri1[k:k + 1, :] * G
            GR2 = GR2 + Ri2[k:k + 1, :] * G
            GR3 = GR3 + Ri3[k:k + 1, :] * G

    invK = 1.0 / K
    GR0 = GR0 * invK
    GR1 = GR1 * invK
    GR2 = GR2 * invK
    GR3 = GR3 * invK

    dq = []
    for q in range(M2):
        dq.append(GR0 * GR0[q:q + 1, :] + GR1 * GR1[q:q + 1, :]
                  + GR2 * GR2[q:q + 1, :] + GR3 * GR3[q:q + 1, :])
    D64 = jnp.concatenate(dq + [ones], axis=0)

    f0 = jnp.tanh(jnp.dot(fw0_ref[...], D64, preferred_element_type=f32))
    f0a = jnp.concatenate([f0, ones], axis=0)
    f1 = jnp.tanh(jnp.dot(fw1_ref[...], f0a, preferred_element_type=f32)) + f0
    f1a = jnp.concatenate([f1, ones], axis=0)
    Ei = jnp.dot(fw2_ref[...], f1a, preferred_element_type=f32) \
        + ENERGY_SHIFT
    out_ref[...] = Ei


def kernel(list_neigh, Imagetype_map, atom_type, ImageDR, nghost, davg, dstd,
           ew0, eb0, ew1, eb1, ew2, eb2, fw0, fb0, fw1, fb1, fw2, fb2):
    B, N, K, _ = ImageDR.shape
    drt = jnp.transpose(ImageDR[0], (2, 1, 0))
    ng = list_neigh.reshape(N, K)

    davgT = davg.reshape(K, 4).T.reshape(4, K, 1)
    drstdT = (1.0 / dstd).reshape(K, 4).T.reshape(4, K, 1)

    GRP = 8
    eye = jnp.eye(GRP, dtype=jnp.float32)
    ew0c = ew0.reshape(8, 1)
    eb0c = eb0.reshape(8, 1)
    ew1T = jnp.concatenate(
        [jnp.kron(eye, ew1.T),
         jnp.tile(eb1, GRP).reshape(16 * GRP, 1)], axis=1)
    eb1c = eb1.reshape(16, 1)
    ew2T = jnp.concatenate(
        [jnp.kron(eye, ew2.T),
         jnp.tile(eb2, GRP).reshape(16 * GRP, 1)], axis=1)
    eb2c = eb2.reshape(16, 1)
    fw0T = jnp.concatenate(
        [fw0.reshape(16, 4, 32).transpose(2, 1, 0).reshape(32, 64),
         fb0.reshape(32, 1)], axis=1)
    fb0c = fb0.reshape(32, 1)
    fw1T = jnp.concatenate([fw1.T, fb1.reshape(32, 1)], axis=1)
    fb1c = fb1.reshape(32, 1)
    fw2T = jnp.concatenate([fw2.T, fb2.reshape(1, 1)], axis=1)
    fb2c = fb2.reshape(1, 1)

    grid = (pl.cdiv(N, TILE),)
    rep2 = lambda a: pl.BlockSpec(a.shape, lambda i: (0, 0))
    rep3 = lambda a: pl.BlockSpec(a.shape, lambda i: (0, 0, 0))

    ei = pl.pallas_call(
        functools.partial(_dp_tile_kernel, K=K),
        grid=grid,
        in_specs=[
            pl.BlockSpec((4, K, TILE), lambda i: (0, 0, i)),
            pl.BlockSpec((TILE, K), lambda i: (i, 0)),
            rep3(davgT), rep3(drstdT),
            rep2(ew0c), rep2(eb0c), rep2(ew1T), rep2(eb1c),
            rep2(ew2T), rep2(eb2c), rep2(fw0T), rep2(fb0c),
            rep2(fw1T), rep2(fb1c), rep2(fw2T), rep2(fb2c),
        ],
        out_specs=pl.BlockSpec((1, TILE), lambda i: (0, i)),
        out_shape=jax.ShapeDtypeStruct((1, N), jnp.float32),
    )(drt, ng, davgT, drstdT, ew0c, eb0c, ew1T, eb1c, ew2T, eb2c,
      fw0T, fb0c, fw1T, fb1c, fw2T, fb2c)

    Ei = ei.reshape(B, N)
    Etot = jnp.sum(Ei, axis=1, keepdims=True)
    return Etot, Ei


# speedup vs baseline: 1.3657x; 1.1126x over previous
import functools

import jax
import jax.numpy as jnp
from jax.experimental import pallas as pl

RMIN = 0.5
RMAX = 6.0
M2 = 4
ENERGY_SHIFT = -3.37

TILE = 2048


def _dp_tile_kernel(drt_ref, ngt_ref, davg_ref, drstd_ref,
                    ew0_ref, eb0_ref, ew1_ref, eb1_ref, ew2_ref, eb2_ref,
                    fw0_ref, fb0_ref, fw1_ref, fb1_ref, fw2_ref, fb2_ref,
                    out_ref, *, K):
    f32 = jnp.float32
    r = drt_ref[0]
    valid = (ngt_ref[...] > 0) & (r > 1e-6)
    r_safe = jnp.where(valid, r, 1.0)
    uu = (r_safe - RMIN) * (1.0 / (RMAX - RMIN))
    poly = uu ** 3 * (-6.0 * uu ** 2 + 15.0 * uu - 10.0) + 1.0
    s = jnp.where(r_safe < RMIN, 1.0 / r_safe,
                  jnp.where(r_safe < RMAX, poly / r_safe, 0.0))
    s = jnp.where(valid, s, 0.0)
    sr = s / r_safe

    Ri0 = (s - davg_ref[0]) * drstd_ref[0]
    Ri1 = (sr * drt_ref[1] - davg_ref[1]) * drstd_ref[1]
    Ri2 = (sr * drt_ref[2] - davg_ref[2]) * drstd_ref[2]
    Ri3 = (sr * drt_ref[3] - davg_ref[3]) * drstd_ref[3]

    ew1T = ew1_ref[...]
    ew2T = ew2_ref[...]
    eb1c = eb1_ref[...]
    eb2c = eb2_ref[...]
    ew0c = ew0_ref[...]
    eb0c = eb0_ref[...]

    T = r.shape[1]
    GR0 = jnp.zeros((16, T), f32)
    GR1 = jnp.zeros((16, T), f32)
    GR2 = jnp.zeros((16, T), f32)
    GR3 = jnp.zeros((16, T), f32)
    ones = jnp.ones((1, T), f32)
    GRP = 8
    for g in range(K // GRP):
        ks = range(g * GRP, (g + 1) * GRP)
        h0s = [jnp.tanh(ew0c * Ri0[k:k + 1, :] + eb0c) for k in ks]
        resid = jnp.concatenate([h for h0 in h0s for h in (h0, h0)], axis=0)
        H0a = jnp.concatenate(h0s + [ones], axis=0)
        H1 = jnp.tanh(jnp.dot(ew1T, H0a, preferred_element_type=f32)) + resid
        H1a = jnp.concatenate([H1, ones], axis=0)
        Gg = jnp.tanh(jnp.dot(ew2T, H1a, preferred_element_type=f32)) + H1
        for j, k in enumerate(ks):
            G = Gg[16 * j:16 * (j + 1), :]
            GR0 = GR0 + Ri0[k:k + 1, :] * G
            GR1 = GR1 + Ri1[k:k + 1, :] * G
            GR2 = GR2 + Ri2[k:k + 1, :] * G
            GR3 = GR3 + Ri3[k:k + 1, :] * G

    invK = 1.0 / K
    GR0 = GR0 * invK
    GR1 = GR1 * invK
    GR2 = GR2 * invK
    GR3 = GR3 * invK

    dq = []
    for q in range(M2):
        dq.append(GR0 * GR0[q:q + 1, :] + GR1 * GR1[q:q + 1, :]
                  + GR2 * GR2[q:q + 1, :] + GR3 * GR3[q:q + 1, :])
    D64 = jnp.concatenate(dq + [ones], axis=0)

    f0 = jnp.tanh(jnp.dot(fw0_ref[...], D64, preferred_element_type=f32))
    f0a = jnp.concatenate([f0, ones], axis=0)
    f1 = jnp.tanh(jnp.dot(fw1_ref[...], f0a, preferred_element_type=f32)) + f0
    f1a = jnp.concatenate([f1, ones], axis=0)
    Ei = jnp.dot(fw2_ref[...], f1a, preferred_element_type=f32) \
        + ENERGY_SHIFT
    out_ref[...] = Ei


def kernel(list_neigh, Imagetype_map, atom_type, ImageDR, nghost, davg, dstd,
           ew0, eb0, ew1, eb1, ew2, eb2, fw0, fb0, fw1, fb1, fw2, fb2):
    B, N, K, _ = ImageDR.shape
    drt = jnp.transpose(ImageDR[0], (2, 1, 0))
    ngt = jnp.transpose(list_neigh[0], (1, 0))

    davgT = davg.reshape(K, 4).T.reshape(4, K, 1)
    drstdT = (1.0 / dstd).reshape(K, 4).T.reshape(4, K, 1)

    GRP = 8
    eye = jnp.eye(GRP, dtype=jnp.float32)
    ew0c = ew0.reshape(8, 1)
    eb0c = eb0.reshape(8, 1)
    ew1T = jnp.concatenate(
        [jnp.kron(eye, ew1.T),
         jnp.tile(eb1, GRP).reshape(16 * GRP, 1)], axis=1)
    eb1c = eb1.reshape(16, 1)
    ew2T = jnp.concatenate(
        [jnp.kron(eye, ew2.T),
         jnp.tile(eb2, GRP).reshape(16 * GRP, 1)], axis=1)
    eb2c = eb2.reshape(16, 1)
    fw0T = jnp.concatenate(
        [fw0.reshape(16, 4, 32).transpose(2, 1, 0).reshape(32, 64),
         fb0.reshape(32, 1)], axis=1)
    fb0c = fb0.reshape(32, 1)
    fw1T = jnp.concatenate([fw1.T, fb1.reshape(32, 1)], axis=1)
    fb1c = fb1.reshape(32, 1)
    fw2T = jnp.concatenate([fw2.T, fb2.reshape(1, 1)], axis=1)
    fb2c = fb2.reshape(1, 1)

    grid = (pl.cdiv(N, TILE),)
    rep2 = lambda a: pl.BlockSpec(a.shape, lambda i: (0, 0))
    rep3 = lambda a: pl.BlockSpec(a.shape, lambda i: (0, 0, 0))

    ei = pl.pallas_call(
        functools.partial(_dp_tile_kernel, K=K),
        grid=grid,
        in_specs=[
            pl.BlockSpec((4, K, TILE), lambda i: (0, 0, i)),
            pl.BlockSpec((K, TILE), lambda i: (0, i)),
            rep3(davgT), rep3(drstdT),
            rep2(ew0c), rep2(eb0c), rep2(ew1T), rep2(eb1c),
            rep2(ew2T), rep2(eb2c), rep2(fw0T), rep2(fb0c),
            rep2(fw1T), rep2(fb1c), rep2(fw2T), rep2(fb2c),
        ],
        out_specs=pl.BlockSpec((1, TILE), lambda i: (0, i)),
        out_shape=jax.ShapeDtypeStruct((1, N), jnp.float32),
    )(drt, ngt, davgT, drstdT, ew0c, eb0c, ew1T, eb1c, ew2T, eb2c,
      fw0T, fb0c, fw1T, fb1c, fw2T, fb2c)

    Ei = ei.reshape(B, N)
    Etot = jnp.sum(Ei, axis=1, keepdims=True)
    return Etot, Ei


# speedup vs baseline: 1.3677x; 1.0014x over previous
import functools

import jax
import jax.numpy as jnp
from jax.experimental import pallas as pl

RMIN = 0.5
RMAX = 6.0
M2 = 4
ENERGY_SHIFT = -3.37

TILE = 2048


def _dp_tile_kernel(drt_ref, ngt_ref, davg_ref, drstd_ref,
                    ew0_ref, eb0_ref, ew1_ref, eb1_ref, ew2_ref, eb2_ref,
                    fw0_ref, fb0_ref, fw1_ref, fb1_ref, fw2_ref, fb2_ref,
                    out_ref, acc_ref, *, K, N):
    f32 = jnp.float32
    r = drt_ref[0]
    valid = (ngt_ref[...] > 0) & (r > 1e-6)
    r_safe = jnp.where(valid, r, 1.0)
    uu = (r_safe - RMIN) * (1.0 / (RMAX - RMIN))
    poly = uu ** 3 * (-6.0 * uu ** 2 + 15.0 * uu - 10.0) + 1.0
    s = jnp.where(r_safe < RMIN, 1.0 / r_safe,
                  jnp.where(r_safe < RMAX, poly / r_safe, 0.0))
    s = jnp.where(valid, s, 0.0)
    sr = s / r_safe

    Ri0 = (s - davg_ref[0]) * drstd_ref[0]
    Ri1 = (sr * drt_ref[1] - davg_ref[1]) * drstd_ref[1]
    Ri2 = (sr * drt_ref[2] - davg_ref[2]) * drstd_ref[2]
    Ri3 = (sr * drt_ref[3] - davg_ref[3]) * drstd_ref[3]

    ew1T = ew1_ref[...]
    ew2T = ew2_ref[...]
    eb1c = eb1_ref[...]
    eb2c = eb2_ref[...]
    ew0c = ew0_ref[...]
    eb0c = eb0_ref[...]

    T = r.shape[1]
    GR0 = jnp.zeros((16, T), f32)
    GR1 = jnp.zeros((16, T), f32)
    GR2 = jnp.zeros((16, T), f32)
    GR3 = jnp.zeros((16, T), f32)
    ones = jnp.ones((1, T), f32)
    GRP = 8
    for g in range(K // GRP):
        ks = range(g * GRP, (g + 1) * GRP)
        h0s = [jnp.tanh(ew0c * Ri0[k:k + 1, :] + eb0c) for k in ks]
        resid = jnp.concatenate([h for h0 in h0s for h in (h0, h0)], axis=0)
        H0a = jnp.concatenate(h0s + [ones], axis=0)
        H1 = jnp.tanh(jnp.dot(ew1T, H0a, preferred_element_type=f32)) + resid
        H1a = jnp.concatenate([H1, ones], axis=0)
        Gg = jnp.tanh(jnp.dot(ew2T, H1a, preferred_element_type=f32)) + H1
        for j, k in enumerate(ks):
            G = Gg[16 * j:16 * (j + 1), :]
            GR0 = GR0 + Ri0[k:k + 1, :] * G
            GR1 = GR1 + Ri1[k:k + 1, :] * G
            GR2 = GR2 + Ri2[k:k + 1, :] * G
            GR3 = GR3 + Ri3[k:k + 1, :] * G

    invK = 1.0 / K
    GR0 = GR0 * invK
    GR1 = GR1 * invK
    GR2 = GR2 * invK
    GR3 = GR3 * invK

    dq = []
    for q in range(M2):
        dq.append(GR0 * GR0[q:q + 1, :] + GR1 * GR1[q:q + 1, :]
                  + GR2 * GR2[q:q + 1, :] + GR3 * GR3[q:q + 1, :])
    D64 = jnp.concatenate(dq + [ones], axis=0)

    f0 = jnp.tanh(jnp.dot(fw0_ref[...], D64, preferred_element_type=f32))
    f0a = jnp.concatenate([f0, ones], axis=0)
    f1 = jnp.tanh(jnp.dot(fw1_ref[...], f0a, preferred_element_type=f32)) + f0
    f1a = jnp.concatenate([f1, ones], axis=0)
    Ei = jnp.dot(fw2_ref[...], f1a, preferred_element_type=f32) \
        + ENERGY_SHIFT
    out_ref[...] = Ei

    i = pl.program_id(0)
    lane = jax.lax.broadcasted_iota(jnp.int32, (1, T), 1)
    part = jnp.sum(jnp.where(i * T + lane < N, Ei, 0.0),
                   axis=1, keepdims=True)

    @pl.when(i == 0)
    def _init():
        acc_ref[...] = part

    @pl.when(i > 0)
    def _acc():
        acc_ref[...] = acc_ref[...] + part


def kernel(list_neigh, Imagetype_map, atom_type, ImageDR, nghost, davg, dstd,
           ew0, eb0, ew1, eb1, ew2, eb2, fw0, fb0, fw1, fb1, fw2, fb2):
    B, N, K, _ = ImageDR.shape
    drt = jnp.transpose(ImageDR[0], (2, 1, 0))
    ngt = jnp.transpose(list_neigh[0], (1, 0))

    davgT = davg.reshape(K, 4).T.reshape(4, K, 1)
    drstdT = (1.0 / dstd).reshape(K, 4).T.reshape(4, K, 1)

    GRP = 8
    eye = jnp.eye(GRP, dtype=jnp.float32)
    ew0c = ew0.reshape(8, 1)
    eb0c = eb0.reshape(8, 1)
    ew1T = jnp.concatenate(
        [jnp.kron(eye, ew1.T),
         jnp.tile(eb1, GRP).reshape(16 * GRP, 1)], axis=1)
    eb1c = eb1.reshape(16, 1)
    ew2T = jnp.concatenate(
        [jnp.kron(eye, ew2.T),
         jnp.tile(eb2, GRP).reshape(16 * GRP, 1)], axis=1)
    eb2c = eb2.reshape(16, 1)
    fw0T = jnp.concatenate(
        [fw0.reshape(16, 4, 32).transpose(2, 1, 0).reshape(32, 64),
         fb0.reshape(32, 1)], axis=1)
    fb0c = fb0.reshape(32, 1)
    fw1T = jnp.concatenate([fw1.T, fb1.reshape(32, 1)], axis=1)
    fb1c = fb1.reshape(32, 1)
    fw2T = jnp.concatenate([fw2.T, fb2.reshape(1, 1)], axis=1)
    fb2c = fb2.reshape(1, 1)

    grid = (pl.cdiv(N, TILE),)
    rep2 = lambda a: pl.BlockSpec(a.shape, lambda i: (0, 0))
    rep3 = lambda a: pl.BlockSpec(a.shape, lambda i: (0, 0, 0))

    ei, etot = pl.pallas_call(
        functools.partial(_dp_tile_kernel, K=K, N=N),
        grid=grid,
        in_specs=[
            pl.BlockSpec((4, K, TILE), lambda i: (0, 0, i)),
            pl.BlockSpec((K, TILE), lambda i: (0, i)),
            rep3(davgT), rep3(drstdT),
            rep2(ew0c), rep2(eb0c), rep2(ew1T), rep2(eb1c),
            rep2(ew2T), rep2(eb2c), rep2(fw0T), rep2(fb0c),
            rep2(fw1T), rep2(fb1c), rep2(fw2T), rep2(fb2c),
        ],
        out_specs=[pl.BlockSpec((1, TILE), lambda i: (0, i)),
                   pl.BlockSpec((1, 1), lambda i: (0, 0))],
        out_shape=[jax.ShapeDtypeStruct((1, N), jnp.float32),
                   jax.ShapeDtypeStruct((1, 1), jnp.float32)],
    )(drt, ngt, davgT, drstdT, ew0c, eb0c, ew1T, eb1c, ew2T, eb2c,
      fw0T, fb0c, fw1T, fb1c, fw2T, fb2c)

    Ei = ei.reshape(B, N)
    return etot, Ei


# speedup vs baseline: 1.3742x; 1.0047x over previous
import functools

import jax
import jax.numpy as jnp
from jax.experimental import pallas as pl

RMIN = 0.5
RMAX = 6.0
M2 = 4
ENERGY_SHIFT = -3.37

TILE = 2048


def _dp_tile_kernel(drt_ref, ngt_ref, davg_ref, drstd_ref,
                    ew0_ref, eb0_ref, ew1_ref, ew2_ref,
                    fw0_ref, fw1_ref, fw2_ref,
                    out_ref, acc_ref, *, K, N):
    f32 = jnp.float32
    r = drt_ref[0]
    valid = (ngt_ref[...] > 0) & (r > 1e-6)
    r_safe = jnp.where(valid, r, 1.0)
    uu = (r_safe - RMIN) * (1.0 / (RMAX - RMIN))
    poly = uu ** 3 * (-6.0 * uu ** 2 + 15.0 * uu - 10.0) + 1.0
    s = jnp.where(r_safe < RMIN, 1.0 / r_safe,
                  jnp.where(r_safe < RMAX, poly / r_safe, 0.0))
    s = jnp.where(valid, s, 0.0)
    sr = s / r_safe

    Ri0 = (s - davg_ref[0]) * drstd_ref[0]
    Ri1 = (sr * drt_ref[1] - davg_ref[1]) * drstd_ref[1]
    Ri2 = (sr * drt_ref[2] - davg_ref[2]) * drstd_ref[2]
    Ri3 = (sr * drt_ref[3] - davg_ref[3]) * drstd_ref[3]

    ew1T = ew1_ref[...]
    ew2T = ew2_ref[...]
    ew0c = ew0_ref[...]
    eb0c = eb0_ref[...]

    T = r.shape[1]
    GR0 = jnp.zeros((16, T), f32)
    GR1 = jnp.zeros((16, T), f32)
    GR2 = jnp.zeros((16, T), f32)
    GR3 = jnp.zeros((16, T), f32)
    ones = jnp.ones((1, T), f32)
    GRP = 8
    for g in range(K // GRP):
        ks = range(g * GRP, (g + 1) * GRP)
        h0s = [jnp.tanh(ew0c * Ri0[k:k + 1, :] + eb0c) for k in ks]
        resid = jnp.concatenate([h for h0 in h0s for h in (h0, h0)], axis=0)
        H0a = jnp.concatenate(h0s + [ones], axis=0)
        H1 = jnp.tanh(jnp.dot(ew1T, H0a, preferred_element_type=f32)) + resid
        H1a = jnp.concatenate([H1, ones], axis=0)
        Gg = jnp.tanh(jnp.dot(ew2T, H1a, preferred_element_type=f32)) + H1
        for j, k in enumerate(ks):
            G = Gg[16 * j:16 * (j + 1), :]
            GR0 = GR0 + Ri0[k:k + 1, :] * G
            GR1 = GR1 + Ri1[k:k + 1, :] * G
            GR2 = GR2 + Ri2[k:k + 1, :] * G
            GR3 = GR3 + Ri3[k:k + 1, :] * G

    invK = 1.0 / K
    GR0 = GR0 * invK
    GR1 = GR1 * invK
    GR2 = GR2 * invK
    GR3 = GR3 * invK

    dq = []
    for q in range(M2):
        dq.append(GR0 * GR0[q:q + 1, :] + GR1 * GR1[q:q + 1, :]
                  + GR2 * GR2[q:q + 1, :] + GR3 * GR3[q:q + 1, :])
    D64 = jnp.concatenate(dq + [ones], axis=0)

    f0 = jnp.tanh(jnp.dot(fw0_ref[...], D64, preferred_element_type=f32))
    f0a = jnp.concatenate([f0, ones], axis=0)
    f1 = jnp.tanh(jnp.dot(fw1_ref[...], f0a, preferred_element_type=f32)) + f0
    f1a = jnp.concatenate([f1, ones], axis=0)
    Ei = jnp.dot(fw2_ref[...], f1a, preferred_element_type=f32) \
        + ENERGY_SHIFT
    out_ref[...] = Ei

    i = pl.program_id(0)
    lane = jax.lax.broadcasted_iota(jnp.int32, (1, T), 1)
    part = jnp.sum(jnp.where(i * T + lane < N, Ei, 0.0),
                   axis=1, keepdims=True)

    @pl.when(i == 0)
    def _init():
        acc_ref[...] = part

    @pl.when(i > 0)
    def _acc():
        acc_ref[...] = acc_ref[...] + part


def kernel(list_neigh, Imagetype_map, atom_type, ImageDR, nghost, davg, dstd,
           ew0, eb0, ew1, eb1, ew2, eb2, fw0, fb0, fw1, fb1, fw2, fb2):
    B, N, K, _ = ImageDR.shape
    drt = jnp.transpose(ImageDR[0], (2, 1, 0))
    ngt = jnp.transpose(list_neigh[0], (1, 0))

    davgT = davg.reshape(K, 4).T.reshape(4, K, 1)
    drstdT = (1.0 / dstd).reshape(K, 4).T.reshape(4, K, 1)

    GRP = 8
    eye = jnp.eye(GRP, dtype=jnp.float32)
    ew0c = ew0.reshape(8, 1)
    eb0c = eb0.reshape(8, 1)
    ew1T = jnp.concatenate(
        [jnp.kron(eye, ew1.T),
         jnp.tile(eb1, GRP).reshape(16 * GRP, 1)], axis=1)
    ew2T = jnp.concatenate(
        [jnp.kron(eye, ew2.T),
         jnp.tile(eb2, GRP).reshape(16 * GRP, 1)], axis=1)
    fw0T = jnp.concatenate(
        [fw0.reshape(16, 4, 32).transpose(2, 1, 0).reshape(32, 64),
         fb0.reshape(32, 1)], axis=1)
    fw1T = jnp.concatenate([fw1.T, fb1.reshape(32, 1)], axis=1)
    fw2T = jnp.concatenate([fw2.T, fb2.reshape(1, 1)], axis=1)

    grid = (pl.cdiv(N, TILE),)
    rep2 = lambda a: pl.BlockSpec(a.shape, lambda i: (0, 0))
    rep3 = lambda a: pl.BlockSpec(a.shape, lambda i: (0, 0, 0))

    ei, etot = pl.pallas_call(
        functools.partial(_dp_tile_kernel, K=K, N=N),
        grid=grid,
        in_specs=[
            pl.BlockSpec((4, K, TILE), lambda i: (0, 0, i)),
            pl.BlockSpec((K, TILE), lambda i: (0, i)),
            rep3(davgT), rep3(drstdT),
            rep2(ew0c), rep2(eb0c), rep2(ew1T),
            rep2(ew2T), rep2(fw0T),
            rep2(fw1T), rep2(fw2T),
        ],
        out_specs=[pl.BlockSpec((1, TILE), lambda i: (0, i)),
                   pl.BlockSpec((1, 1), lambda i: (0, 0))],
        out_shape=[jax.ShapeDtypeStruct((1, N), jnp.float32),
                   jax.ShapeDtypeStruct((1, 1), jnp.float32)],
    )(drt, ngt, davgT, drstdT, ew0c, eb0c, ew1T, ew2T,
      fw0T, fw1T, fw2T)

    Ei = ei.reshape(B, N)
    return etot, Ei
